# baseline (speedup 1.0000x reference)
import dataclasses

import jax
import jax.numpy as jnp
from jax import lax
from jax.experimental import pallas as pl
from jax.experimental.pallas import tpu as pltpu
from jax.experimental.pallas import tpu_sc as plsc

_N = 10000
_E = 160000
_D = 256
_EPS = 0.3

_NC = 2
_NS = 16
_L = 16

_Q = 64
_NQ = _D // _Q
_EPT = _E // _NS
_CH = 80
_NCHUNK = _EPT // _CH
_AR = 10000

_mesh = plsc.VectorSubcoreMesh(core_axis_name="c", subcore_axis_name="s")

_sc_params = pltpu.CompilerParams()
if "needs_layout_passes" in pltpu.CompilerParams.__dataclass_fields__:
    _sc_params = dataclasses.replace(_sc_params, needs_layout_passes=False)
if "use_tc_tiling_on_sc" in pltpu.CompilerParams.__dataclass_fields__:
    _sc_params = dataclasses.replace(_sc_params, use_tc_tiling_on_sc=False)


def _tanh_via_exp(a):
    e = jnp.exp(-2.0 * jnp.abs(a))
    t = (1.0 - e) / (1.0 + e)
    return jnp.where(a < 0.0, -t, t)


_ES = _E // (_NC * _NS)


def _deg_body(col_hbm, out_hbm, hist, colv):
    c = lax.axis_index("c")
    s = lax.axis_index("s")
    wid = c * _NS + s

    @pl.loop(0, _N, step=_L)
    def _(i):
        hist[pl.ds(i, _L)] = jnp.zeros((_L,), jnp.float32)

    pltpu.sync_copy(col_hbm.at[pl.ds(wid * _ES, _ES)], colv)
    ones = jnp.ones((_L,), jnp.float32)

    @pl.loop(0, _ES - _L, step=_L)
    def _(i):
        plsc.addupdate_scatter(hist, [colv[pl.ds(i, _L)]], ones)

    tail = colv[pl.ds(_ES - _L, _L)]
    mask = lax.iota(jnp.int32, _L) >= (_L - (_ES % _L or _L))
    plsc.addupdate_scatter(hist, [tail], ones, mask=mask)
    pltpu.sync_copy(hist, out_hbm.at[wid])


def _deg_partials(col):
    return pl.kernel(
        _deg_body,
        out_type=jax.ShapeDtypeStruct((_NC * _NS, _N), jnp.float32),
        mesh=_mesh,
        scratch_types=[
            pltpu.VMEM((_N,), jnp.float32),
            pltpu.VMEM((_ES,), jnp.int32),
        ],
        compiler_params=_sc_params,
    )(col)


_NBUF = 5
_STAG_G = 3
_STAG_W = 2


def _make_edge_body(emit_alpha):
    def body(xs_hbm, row_hbm, col_hbm, al_hbm, ar_hbm, dis_hbm, *rest):
        if emit_alpha:
            out_hbm, alpha_hbm = rest[0], rest[1]
            rest = rest[2:]
        else:
            out_hbm = rest[0]
            rest = rest[1:]
        (al_v, ar_v, dis_v, rowi, radj, coli, coefs, alph,
         rb0, rb1, rb2, rb3, rb4, acc,
         g0, g1, g2, g3, g4, s0, s1, s2, s3, s4) = rest
        rbufs = [rb0, rb1, rb2, rb3, rb4]
        gsem = [g0, g1, g2, g3, g4]
        ssem = [s0, s1, s2, s3, s4]
        c = lax.axis_index("c")
        s = lax.axis_index("s")

        def gather_start(k, b, off):
            for i in range(_CH // _L):
                sl = pl.ds(i * _L, _L)
                radj[b, sl] = rowi[k, sl] + off
            pltpu.async_copy(xs_hbm.at[radj.at[b]], rbufs[b], gsem[b])

        def gather_wait(k, b):
            pltpu.make_async_copy(xs_hbm.at[radj.at[b]], rbufs[b],
                                  gsem[b]).wait()

        def scatter_start(k, b):
            pltpu.async_copy(rbufs[b], acc.at[coli.at[k]], ssem[b],
                             add=True)

        def scatter_wait(k, b):
            pltpu.make_async_copy(rbufs[b], acc.at[coli.at[k]],
                                  ssem[b]).wait()

        def scale(k, b):
            @plsc.parallel_loop(0, _CH, step=2, unroll=8)
            def _(j0):
                for u in range(2):
                    j = j0 + u
                    cs = plsc.load_gather(
                        coefs, [jnp.zeros((_L,), jnp.int32) + (k * _CH + j)]
                    )
                    for d in range(_Q // _L):
                        dsl = pl.ds(d * _L, _L)
                        rbufs[b][j, dsl] = rbufs[b][j, dsl] * cs

        pltpu.sync_copy(al_hbm, al_v)
        pltpu.sync_copy(ar_hbm, ar_v)
        pltpu.sync_copy(dis_hbm, dis_v)
        pltpu.sync_copy(row_hbm.at[s], rowi)
        pltpu.sync_copy(col_hbm.at[s], coli)

        @pl.loop(0, _NCHUNK)
        def _(k):
            @plsc.parallel_loop(0, _CH, step=_L, unroll=5)
            def _(i0):
                sl = pl.ds(i0, _L)
                r = rowi[k, sl]
                cc = coli[k, sl]
                t = _tanh_via_exp(
                    plsc.load_gather(al_v, [r]) + plsc.load_gather(ar_v, [cc])
                )
                cf = (
                    t
                    * plsc.load_gather(dis_v, [r])
                    * plsc.load_gather(dis_v, [cc])
                )
                coefs[pl.ds(k * _CH + i0, _L)] = cf
                if emit_alpha:
                    alph[sl] = t
            if emit_alpha:
                @pl.when(c == 0)
                def _():
                    pltpu.sync_copy(
                        alph, alpha_hbm.at[pl.ds(s * _EPT + k * _CH, _CH)]
                    )

        for q in range(2):
            qg = c * 2 + q

            @pl.loop(0, _CH)
            def _(j):
                for d in range(_Q // _L):
                    rbufs[0][j, pl.ds(d * _L, _L)] = jnp.zeros(
                        (_L,), jnp.float32
                    )

            @pl.when(s < _NS - 1)
            def _():
                for t in range(7):
                    pltpu.sync_copy(
                        rbufs[0], acc.at[pl.ds(s * 624 + t * 80, 80)]
                    )
                pltpu.sync_copy(
                    rbufs[0].at[pl.ds(0, 64)],
                    acc.at[pl.ds(s * 624 + 560, 64)],
                )

            @pl.when(s == _NS - 1)
            def _():
                for t in range(8):
                    pltpu.sync_copy(rbufs[0], acc.at[pl.ds(9360 + t * 80, 80)])

            off = qg * _N

            for b in range(_STAG_G):
                gather_start(b, b, off)
            plsc.subcore_barrier()

            @pl.loop(0, _NCHUNK, step=_NBUF)
            def _(m):
                for b in range(_NBUF):
                    j = m + b
                    bn = (b + _STAG_G) % _NBUF
                    gather_wait(j, b)
                    scale(j, b)
                    scatter_start(j, b)
                    @pl.when(j >= _STAG_W)
                    def _():
                        scatter_wait(j - _STAG_W, bn)
                    @pl.when(j <= _NCHUNK - 1 - _STAG_G)
                    def _():
                        gather_start(j + _STAG_G, bn, off)

            scatter_wait(_NCHUNK - 2, (_NCHUNK - 2) % _NBUF)
            scatter_wait(_NCHUNK - 1, (_NCHUNK - 1) % _NBUF)

            plsc.subcore_barrier()

            @pl.when(s < _NS - 1)
            def _():
                pltpu.sync_copy(
                    acc.at[pl.ds(s * 624, 624)],
                    out_hbm.at[qg, pl.ds(s * 624, 624)],
                )

            @pl.when(s == _NS - 1)
            def _():
                pltpu.sync_copy(
                    acc.at[pl.ds(9360, 640)], out_hbm.at[qg, pl.ds(9360, 640)]
                )

    return body


def _make_edge_call(emit_alpha):
    out_type = [jax.ShapeDtypeStruct((_NQ, _N, _Q), jnp.float32)]
    if emit_alpha:
        out_type.append(jax.ShapeDtypeStruct((_E,), jnp.float32))
    scratch = [
        pltpu.VMEM((_N,), jnp.float32),
        pltpu.VMEM((_N,), jnp.float32),
        pltpu.VMEM((_N,), jnp.float32),
        pltpu.VMEM((_NCHUNK, _CH), jnp.int32),
        pltpu.VMEM((_NBUF, _CH), jnp.int32),
        pltpu.VMEM((_NCHUNK, _CH), jnp.int32),
        pltpu.VMEM((_EPT,), jnp.float32),
        pltpu.VMEM((_CH,), jnp.float32),
    ] + [pltpu.VMEM((_CH, _Q), jnp.float32)] * _NBUF + [
        pltpu.VMEM_SHARED((_AR, _Q), jnp.float32),
    ] + [pltpu.SemaphoreType.DMA] * (2 * _NBUF)
    body = _make_edge_body(emit_alpha)

    def call(xs, rows3, cols3, al, ar, dis):
        return pl.kernel(
            body,
            out_type=tuple(out_type) if emit_alpha else out_type[0],
            mesh=_mesh,
            scratch_types=scratch,
            compiler_params=_sc_params,
        )(xs, rows3, cols3, al, ar, dis)

    return call


_edge_plain = _make_edge_call(False)
_edge_alpha = _make_edge_call(True)


def _pre_body(x_ref, wl_ref, wr_ref, degp_ref, al_ref, ar_ref, dis_ref):
    x = x_ref[...]
    al_ref[...] = x @ wl_ref[...]
    ar_ref[...] = x @ wr_ref[...]
    deg = jnp.sum(degp_ref[...], axis=0) + 1.0
    dis_ref[...] = lax.rsqrt(deg)


def _pre(x, wl, wr, degp):
    return pl.pallas_call(
        _pre_body,
        out_shape=(jax.ShapeDtypeStruct((_N,), jnp.float32),) * 3,
    )(x, wl, wr, degp)


_BL = 1000
_NB = _N // _BL


def _layer_body(s_ref, xin_ref, x0_ref, al_ref, ar_ref, dis_ref, wl_ref, wr_ref,
                h_ref, h4_ref, t_ref, aln_ref, arn_ref):
    sc = jnp.concatenate([s_ref[i] for i in range(_NQ)], axis=1)
    t = jnp.tanh(al_ref[0, 0] + ar_ref[0, 0])
    dis = dis_ref[0, 0]
    cf = t * dis * dis
    h = sc + cf[:, None] * xin_ref[...] + _EPS * x0_ref[...]
    h = jnp.maximum(h, 0.0)
    h_ref[...] = h
    for i in range(_NQ):
        h4_ref[i] = h[:, i * _Q:(i + 1) * _Q]
    t_ref[0, 0] = t
    aln_ref[0, 0] = h @ wl_ref[0]
    arn_ref[0, 0] = h @ wr_ref[0]


def _layer(s, xin, x0, al, ar, dis, wl, wr):
    vec_spec = pl.BlockSpec((1, 1, _BL), lambda i: (i, 0, 0))
    mat_spec = pl.BlockSpec((_BL, _D), lambda i: (i, 0))
    h, h4, t, aln, arn = pl.pallas_call(
        _layer_body,
        grid=(_NB,),
        in_specs=[
            pl.BlockSpec((_NQ, _BL, _Q), lambda i: (0, i, 0)),
            mat_spec,
            mat_spec,
            vec_spec,
            vec_spec,
            vec_spec,
            pl.BlockSpec((1, _D), lambda i: (0, 0)),
            pl.BlockSpec((1, _D), lambda i: (0, 0)),
        ],
        out_specs=(
            mat_spec,
            pl.BlockSpec((_NQ, _BL, _Q), lambda i: (0, i, 0)),
            vec_spec,
            vec_spec,
            vec_spec,
        ),
        out_shape=(
            jax.ShapeDtypeStruct((_N, _D), jnp.float32),
            jax.ShapeDtypeStruct((_NQ, _N, _Q), jnp.float32),
            jax.ShapeDtypeStruct((_NB, 1, _BL), jnp.float32),
            jax.ShapeDtypeStruct((_NB, 1, _BL), jnp.float32),
            jax.ShapeDtypeStruct((_NB, 1, _BL), jnp.float32),
        ),
    )(
        s, xin, x0,
        al.reshape(_NB, 1, _BL), ar.reshape(_NB, 1, _BL),
        dis.reshape(_NB, 1, _BL),
        wl.reshape(1, _D), wr.reshape(1, _D),
    )
    return h, h4, t.reshape(_N), aln.reshape(_N), arn.reshape(_N)


def kernel(x, edges_index, pertub, w_l0, w_r0, w_l1, w_r1):
    row = edges_index[0]
    col = edges_index[1]
    rows3 = row.reshape(_NS, _NCHUNK, _CH)
    cols3 = col.reshape(_NS, _NCHUNK, _CH)
    x4 = jnp.concatenate(
        [x[:, i * _Q:(i + 1) * _Q] for i in range(_NQ)], axis=0
    )

    degp = _deg_partials(col)
    al0, ar0, dis = _pre(x, w_l0, w_r0, degp)
    s0 = _edge_plain(x4, rows3, cols3, al0, ar0, dis)
    h1, h14, _t0, al1, ar1 = _layer(s0, x, x, al0, ar0, dis, w_l1, w_r1)
    h14 = h14.reshape(_NQ * _N, _Q)
    s1, alpha_e = _edge_alpha(h14, rows3, cols3, al1, ar1, dis)
    h2, _h24, alpha_n, _a, _b = _layer(s1, h1, x, al1, ar1, dis, w_l1, w_r1)

    loop = jnp.arange(_N, dtype=edges_index.dtype)
    edge_index_sl = jnp.stack(
        [jnp.concatenate([row, loop]), jnp.concatenate([col, loop])]
    )
    alpha = jnp.concatenate([alpha_e, alpha_n])
    return h2, edge_index_sl, alpha


# speedup vs baseline: 1.0110x; 1.0110x over previous
import dataclasses

import jax
import jax.numpy as jnp
from jax import lax
from jax.experimental import pallas as pl
from jax.experimental.pallas import tpu as pltpu
from jax.experimental.pallas import tpu_sc as plsc

_N = 10000
_E = 160000
_D = 256
_EPS = 0.3

_NC = 2
_NS = 16
_L = 16

_Q = 64
_NQ = _D // _Q
_EPT = _E // _NS
_CH = 80
_NCHUNK = _EPT // _CH
_AR = 10000

_mesh = plsc.VectorSubcoreMesh(core_axis_name="c", subcore_axis_name="s")

_sc_params = pltpu.CompilerParams()
if "needs_layout_passes" in pltpu.CompilerParams.__dataclass_fields__:
    _sc_params = dataclasses.replace(_sc_params, needs_layout_passes=False)
if "use_tc_tiling_on_sc" in pltpu.CompilerParams.__dataclass_fields__:
    _sc_params = dataclasses.replace(_sc_params, use_tc_tiling_on_sc=False)


def _tanh_via_exp(a):
    e = jnp.exp(-2.0 * jnp.abs(a))
    t = (1.0 - e) / (1.0 + e)
    return jnp.where(a < 0.0, -t, t)


_ES = _E // (_NC * _NS)


def _deg_body(col_hbm, out_hbm, hist, colv):
    c = lax.axis_index("c")
    s = lax.axis_index("s")
    wid = c * _NS + s

    @pl.loop(0, _N, step=_L)
    def _(i):
        hist[pl.ds(i, _L)] = jnp.zeros((_L,), jnp.float32)

    pltpu.sync_copy(col_hbm.at[pl.ds(wid * _ES, _ES)], colv)
    ones = jnp.ones((_L,), jnp.float32)

    @pl.loop(0, _ES - _L, step=_L)
    def _(i):
        plsc.addupdate_scatter(hist, [colv[pl.ds(i, _L)]], ones)

    tail = colv[pl.ds(_ES - _L, _L)]
    mask = lax.iota(jnp.int32, _L) >= (_L - (_ES % _L or _L))
    plsc.addupdate_scatter(hist, [tail], ones, mask=mask)
    pltpu.sync_copy(hist, out_hbm.at[wid])


def _deg_partials(col):
    return pl.kernel(
        _deg_body,
        out_type=jax.ShapeDtypeStruct((_NC * _NS, _N), jnp.float32),
        mesh=_mesh,
        scratch_types=[
            pltpu.VMEM((_N,), jnp.float32),
            pltpu.VMEM((_ES,), jnp.int32),
        ],
        compiler_params=_sc_params,
    )(col)


_NBUF = 5
_STAG_G = 3
_STAG_W = 2


def _make_edge_body(emit_alpha):
    def body(xs_hbm, row_hbm, col_hbm, al_hbm, ar_hbm, dis_hbm, *rest):
        if emit_alpha:
            out_hbm, alpha_hbm = rest[0], rest[1]
            rest = rest[2:]
        else:
            out_hbm = rest[0]
            rest = rest[1:]
        (al_v, ar_v, dis_v, rowi, radj, coli, coefs, alph,
         rb0, rb1, rb2, rb3, rb4, acc,
         g0, g1, g2, g3, g4, s0, s1, s2, s3, s4) = rest
        rbufs = [rb0, rb1, rb2, rb3, rb4]
        gsem = [g0, g1, g2, g3, g4]
        ssem = [s0, s1, s2, s3, s4]
        c = lax.axis_index("c")
        s = lax.axis_index("s")

        def gather_start(k, b, off):
            for i in range(_CH // _L):
                sl = pl.ds(i * _L, _L)
                radj[b, sl] = rowi[k, sl] + off
            pltpu.async_copy(xs_hbm.at[radj.at[b]], rbufs[b], gsem[b])

        def gather_wait(k, b):
            pltpu.make_async_copy(xs_hbm.at[radj.at[b]], rbufs[b],
                                  gsem[b]).wait()

        def scatter_start(k, b):
            pltpu.async_copy(rbufs[b], acc.at[coli.at[k]], ssem[b],
                             add=True)

        def scatter_wait(k, b):
            pltpu.make_async_copy(rbufs[b], acc.at[coli.at[k]],
                                  ssem[b]).wait()

        def scale(k, b):
            @plsc.parallel_loop(0, _CH, step=1, unroll=8)
            def _(j0):
                for u in range(1):
                    j = j0 + u
                    cs = plsc.load_gather(
                        coefs, [jnp.zeros((_L,), jnp.int32) + (k * _CH + j)]
                    )
                    for d in range(_Q // _L):
                        dsl = pl.ds(d * _L, _L)
                        rbufs[b][j, dsl] = rbufs[b][j, dsl] * cs

        pltpu.sync_copy(al_hbm, al_v)
        pltpu.sync_copy(ar_hbm, ar_v)
        pltpu.sync_copy(dis_hbm, dis_v)
        pltpu.sync_copy(row_hbm.at[s], rowi)
        pltpu.sync_copy(col_hbm.at[s], coli)

        @pl.loop(0, _NCHUNK)
        def _(k):
            @plsc.parallel_loop(0, _CH, step=_L, unroll=5)
            def _(i0):
                sl = pl.ds(i0, _L)
                r = rowi[k, sl]
                cc = coli[k, sl]
                t = _tanh_via_exp(
                    plsc.load_gather(al_v, [r]) + plsc.load_gather(ar_v, [cc])
                )
                cf = (
                    t
                    * plsc.load_gather(dis_v, [r])
                    * plsc.load_gather(dis_v, [cc])
                )
                coefs[pl.ds(k * _CH + i0, _L)] = cf
                if emit_alpha:
                    alph[sl] = t
            if emit_alpha:
                @pl.when(c == 0)
                def _():
                    pltpu.sync_copy(
                        alph, alpha_hbm.at[pl.ds(s * _EPT + k * _CH, _CH)]
                    )

        for q in range(2):
            qg = c * 2 + q

            @pl.loop(0, _CH)
            def _(j):
                for d in range(_Q // _L):
                    rbufs[0][j, pl.ds(d * _L, _L)] = jnp.zeros(
                        (_L,), jnp.float32
                    )

            @pl.when(s < _NS - 1)
            def _():
                for t in range(7):
                    pltpu.sync_copy(
                        rbufs[0], acc.at[pl.ds(s * 624 + t * 80, 80)]
                    )
                pltpu.sync_copy(
                    rbufs[0].at[pl.ds(0, 64)],
                    acc.at[pl.ds(s * 624 + 560, 64)],
                )

            @pl.when(s == _NS - 1)
            def _():
                for t in range(8):
                    pltpu.sync_copy(rbufs[0], acc.at[pl.ds(9360 + t * 80, 80)])

            off = qg * _N

            for b in range(_STAG_G):
                gather_start(b, b, off)
            plsc.subcore_barrier()

            @pl.loop(0, _NCHUNK, step=_NBUF)
            def _(m):
                for b in range(_NBUF):
                    j = m + b
                    bn = (b + _STAG_G) % _NBUF
                    gather_wait(j, b)
                    scale(j, b)
                    scatter_start(j, b)
                    @pl.when(j >= _STAG_W)
                    def _():
                        scatter_wait(j - _STAG_W, bn)
                    @pl.when(j <= _NCHUNK - 1 - _STAG_G)
                    def _():
                        gather_start(j + _STAG_G, bn, off)

            scatter_wait(_NCHUNK - 2, (_NCHUNK - 2) % _NBUF)
            scatter_wait(_NCHUNK - 1, (_NCHUNK - 1) % _NBUF)

            plsc.subcore_barrier()

            @pl.when(s < _NS - 1)
            def _():
                pltpu.sync_copy(
                    acc.at[pl.ds(s * 624, 624)],
                    out_hbm.at[qg, pl.ds(s * 624, 624)],
                )

            @pl.when(s == _NS - 1)
            def _():
                pltpu.sync_copy(
                    acc.at[pl.ds(9360, 640)], out_hbm.at[qg, pl.ds(9360, 640)]
                )

    return body


def _make_edge_call(emit_alpha):
    out_type = [jax.ShapeDtypeStruct((_NQ, _N, _Q), jnp.float32)]
    if emit_alpha:
        out_type.append(jax.ShapeDtypeStruct((_E,), jnp.float32))
    scratch = [
        pltpu.VMEM((_N,), jnp.float32),
        pltpu.VMEM((_N,), jnp.float32),
        pltpu.VMEM((_N,), jnp.float32),
        pltpu.VMEM((_NCHUNK, _CH), jnp.int32),
        pltpu.VMEM((_NBUF, _CH), jnp.int32),
        pltpu.VMEM((_NCHUNK, _CH), jnp.int32),
        pltpu.VMEM((_EPT,), jnp.float32),
        pltpu.VMEM((_CH,), jnp.float32),
    ] + [pltpu.VMEM((_CH, _Q), jnp.float32)] * _NBUF + [
        pltpu.VMEM_SHARED((_AR, _Q), jnp.float32),
    ] + [pltpu.SemaphoreType.DMA] * (2 * _NBUF)
    body = _make_edge_body(emit_alpha)

    def call(xs, rows3, cols3, al, ar, dis):
        return pl.kernel(
            body,
            out_type=tuple(out_type) if emit_alpha else out_type[0],
            mesh=_mesh,
            scratch_types=scratch,
            compiler_params=_sc_params,
        )(xs, rows3, cols3, al, ar, dis)

    return call


_edge_plain = _make_edge_call(False)
_edge_alpha = _make_edge_call(True)


def _pre_body(x_ref, wl_ref, wr_ref, degp_ref, al_ref, ar_ref, dis_ref):
    x = x_ref[...]
    al_ref[...] = x @ wl_ref[...]
    ar_ref[...] = x @ wr_ref[...]
    deg = jnp.sum(degp_ref[...], axis=0) + 1.0
    dis_ref[...] = lax.rsqrt(deg)


def _pre(x, wl, wr, degp):
    return pl.pallas_call(
        _pre_body,
        out_shape=(jax.ShapeDtypeStruct((_N,), jnp.float32),) * 3,
    )(x, wl, wr, degp)


_BL = 1000
_NB = _N // _BL


def _layer_body(s_ref, xin_ref, x0_ref, al_ref, ar_ref, dis_ref, wl_ref, wr_ref,
                h_ref, h4_ref, t_ref, aln_ref, arn_ref):
    sc = jnp.concatenate([s_ref[i] for i in range(_NQ)], axis=1)
    t = jnp.tanh(al_ref[0, 0] + ar_ref[0, 0])
    dis = dis_ref[0, 0]
    cf = t * dis * dis
    h = sc + cf[:, None] * xin_ref[...] + _EPS * x0_ref[...]
    h = jnp.maximum(h, 0.0)
    h_ref[...] = h
    for i in range(_NQ):
        h4_ref[i] = h[:, i * _Q:(i + 1) * _Q]
    t_ref[0, 0] = t
    aln_ref[0, 0] = h @ wl_ref[0]
    arn_ref[0, 0] = h @ wr_ref[0]


def _layer(s, xin, x0, al, ar, dis, wl, wr):
    vec_spec = pl.BlockSpec((1, 1, _BL), lambda i: (i, 0, 0))
    mat_spec = pl.BlockSpec((_BL, _D), lambda i: (i, 0))
    h, h4, t, aln, arn = pl.pallas_call(
        _layer_body,
        grid=(_NB,),
        in_specs=[
            pl.BlockSpec((_NQ, _BL, _Q), lambda i: (0, i, 0)),
            mat_spec,
            mat_spec,
            vec_spec,
            vec_spec,
            vec_spec,
            pl.BlockSpec((1, _D), lambda i: (0, 0)),
            pl.BlockSpec((1, _D), lambda i: (0, 0)),
        ],
        out_specs=(
            mat_spec,
            pl.BlockSpec((_NQ, _BL, _Q), lambda i: (0, i, 0)),
            vec_spec,
            vec_spec,
            vec_spec,
        ),
        out_shape=(
            jax.ShapeDtypeStruct((_N, _D), jnp.float32),
            jax.ShapeDtypeStruct((_NQ, _N, _Q), jnp.float32),
            jax.ShapeDtypeStruct((_NB, 1, _BL), jnp.float32),
            jax.ShapeDtypeStruct((_NB, 1, _BL), jnp.float32),
            jax.ShapeDtypeStruct((_NB, 1, _BL), jnp.float32),
        ),
    )(
        s, xin, x0,
        al.reshape(_NB, 1, _BL), ar.reshape(_NB, 1, _BL),
        dis.reshape(_NB, 1, _BL),
        wl.reshape(1, _D), wr.reshape(1, _D),
    )
    return h, h4, t.reshape(_N), aln.reshape(_N), arn.reshape(_N)


def kernel(x, edges_index, pertub, w_l0, w_r0, w_l1, w_r1):
    row = edges_index[0]
    col = edges_index[1]
    rows3 = row.reshape(_NS, _NCHUNK, _CH)
    cols3 = col.reshape(_NS, _NCHUNK, _CH)
    x4 = jnp.concatenate(
        [x[:, i * _Q:(i + 1) * _Q] for i in range(_NQ)], axis=0
    )

    degp = _deg_partials(col)
    al0, ar0, dis = _pre(x, w_l0, w_r0, degp)
    s0 = _edge_plain(x4, rows3, cols3, al0, ar0, dis)
    h1, h14, _t0, al1, ar1 = _layer(s0, x, x, al0, ar0, dis, w_l1, w_r1)
    h14 = h14.reshape(_NQ * _N, _Q)
    s1, alpha_e = _edge_alpha(h14, rows3, cols3, al1, ar1, dis)
    h2, _h24, alpha_n, _a, _b = _layer(s1, h1, x, al1, ar1, dis, w_l1, w_r1)

    loop = jnp.arange(_N, dtype=edges_index.dtype)
    edge_index_sl = jnp.stack(
        [jnp.concatenate([row, loop]), jnp.concatenate([col, loop])]
    )
    alpha = jnp.concatenate([alpha_e, alpha_n])
    return h2, edge_index_sl, alpha


# speedup vs baseline: 1.0123x; 1.0013x over previous
import dataclasses

import jax
import jax.numpy as jnp
from jax import lax
from jax.experimental import pallas as pl
from jax.experimental.pallas import tpu as pltpu
from jax.experimental.pallas import tpu_sc as plsc

_N = 10000
_E = 160000
_D = 256
_EPS = 0.3

_NC = 2
_NS = 16
_L = 16

_Q = 64
_NQ = _D // _Q
_EPT = _E // _NS
_CH = 80
_NCHUNK = _EPT // _CH
_AR = 10000

_mesh = plsc.VectorSubcoreMesh(core_axis_name="c", subcore_axis_name="s")

_sc_params = pltpu.CompilerParams()
if "needs_layout_passes" in pltpu.CompilerParams.__dataclass_fields__:
    _sc_params = dataclasses.replace(_sc_params, needs_layout_passes=False)
if "use_tc_tiling_on_sc" in pltpu.CompilerParams.__dataclass_fields__:
    _sc_params = dataclasses.replace(_sc_params, use_tc_tiling_on_sc=False)


def _tanh_via_exp(a):
    e = jnp.exp(-2.0 * jnp.abs(a))
    t = (1.0 - e) / (1.0 + e)
    return jnp.where(a < 0.0, -t, t)


_ES = _E // (_NC * _NS)


def _deg_body(col_hbm, out_hbm, hist, colv):
    c = lax.axis_index("c")
    s = lax.axis_index("s")
    wid = c * _NS + s

    @pl.loop(0, _N, step=_L)
    def _(i):
        hist[pl.ds(i, _L)] = jnp.zeros((_L,), jnp.float32)

    pltpu.sync_copy(col_hbm.at[pl.ds(wid * _ES, _ES)], colv)
    ones = jnp.ones((_L,), jnp.float32)

    @pl.loop(0, _ES - _L, step=_L)
    def _(i):
        plsc.addupdate_scatter(hist, [colv[pl.ds(i, _L)]], ones)

    tail = colv[pl.ds(_ES - _L, _L)]
    mask = lax.iota(jnp.int32, _L) >= (_L - (_ES % _L or _L))
    plsc.addupdate_scatter(hist, [tail], ones, mask=mask)
    pltpu.sync_copy(hist, out_hbm.at[wid])


def _deg_partials(col):
    return pl.kernel(
        _deg_body,
        out_type=jax.ShapeDtypeStruct((_NC * _NS, _N), jnp.float32),
        mesh=_mesh,
        scratch_types=[
            pltpu.VMEM((_N,), jnp.float32),
            pltpu.VMEM((_ES,), jnp.int32),
        ],
        compiler_params=_sc_params,
    )(col)


_NBUF = 5
_STAG_G = 3
_STAG_W = 2


def _make_edge_body(emit_alpha):
    def body(xs_hbm, row_hbm, col_hbm, al_hbm, ar_hbm, dis_hbm, *rest):
        if emit_alpha:
            out_hbm, alpha_hbm = rest[0], rest[1]
            rest = rest[2:]
        else:
            out_hbm = rest[0]
            rest = rest[1:]
        (al_v, ar_v, dis_v, rowi, radj, coli, coefs, alph,
         rb0, rb1, rb2, rb3, rb4, acc,
         g0, g1, g2, g3, g4, s0, s1, s2, s3, s4) = rest
        rbufs = [rb0, rb1, rb2, rb3, rb4]
        gsem = [g0, g1, g2, g3, g4]
        ssem = [s0, s1, s2, s3, s4]
        c = lax.axis_index("c")
        s = lax.axis_index("s")

        def gather_start(k, b, off):
            for i in range(_CH // _L):
                sl = pl.ds(i * _L, _L)
                radj[b, sl] = rowi[k, sl] + off
            pltpu.async_copy(xs_hbm.at[radj.at[b]], rbufs[b], gsem[b])

        def gather_wait(k, b):
            pltpu.make_async_copy(xs_hbm.at[radj.at[b]], rbufs[b],
                                  gsem[b]).wait()

        def scatter_start(k, b):
            pltpu.async_copy(rbufs[b], acc.at[coli.at[k]], ssem[b],
                             add=True)

        def scatter_wait(k, b):
            pltpu.make_async_copy(rbufs[b], acc.at[coli.at[k]],
                                  ssem[b]).wait()

        def scale(k, b):
            @plsc.parallel_loop(0, _CH, step=2, unroll=4)
            def _(j0):
                for u in range(2):
                    j = j0 + u
                    cs = plsc.load_gather(
                        coefs, [jnp.zeros((_L,), jnp.int32) + (k * _CH + j)]
                    )
                    for d in range(_Q // _L):
                        dsl = pl.ds(d * _L, _L)
                        rbufs[b][j, dsl] = rbufs[b][j, dsl] * cs

        pltpu.sync_copy(al_hbm, al_v)
        pltpu.sync_copy(ar_hbm, ar_v)
        pltpu.sync_copy(dis_hbm, dis_v)
        pltpu.sync_copy(row_hbm.at[s], rowi)
        pltpu.sync_copy(col_hbm.at[s], coli)

        @pl.loop(0, _NCHUNK)
        def _(k):
            @plsc.parallel_loop(0, _CH, step=_L, unroll=5)
            def _(i0):
                sl = pl.ds(i0, _L)
                r = rowi[k, sl]
                cc = coli[k, sl]
                t = _tanh_via_exp(
                    plsc.load_gather(al_v, [r]) + plsc.load_gather(ar_v, [cc])
                )
                cf = (
                    t
                    * plsc.load_gather(dis_v, [r])
                    * plsc.load_gather(dis_v, [cc])
                )
                coefs[pl.ds(k * _CH + i0, _L)] = cf
                if emit_alpha:
                    alph[sl] = t
            if emit_alpha:
                @pl.when(c == 0)
                def _():
                    pltpu.sync_copy(
                        alph, alpha_hbm.at[pl.ds(s * _EPT + k * _CH, _CH)]
                    )

        for q in range(2):
            qg = c * 2 + q

            @pl.loop(0, _CH)
            def _(j):
                for d in range(_Q // _L):
                    rbufs[0][j, pl.ds(d * _L, _L)] = jnp.zeros(
                        (_L,), jnp.float32
                    )

            @pl.when(s < _NS - 1)
            def _():
                for t in range(7):
                    pltpu.sync_copy(
                        rbufs[0], acc.at[pl.ds(s * 624 + t * 80, 80)]
                    )
                pltpu.sync_copy(
                    rbufs[0].at[pl.ds(0, 64)],
                    acc.at[pl.ds(s * 624 + 560, 64)],
                )

            @pl.when(s == _NS - 1)
            def _():
                for t in range(8):
                    pltpu.sync_copy(rbufs[0], acc.at[pl.ds(9360 + t * 80, 80)])

            off = qg * _N

            for b in range(_STAG_G):
                gather_start(b, b, off)
            plsc.subcore_barrier()

            @pl.loop(0, _NCHUNK, step=_NBUF)
            def _(m):
                for b in range(_NBUF):
                    j = m + b
                    bn = (b + _STAG_G) % _NBUF
                    gather_wait(j, b)
                    scale(j, b)
                    scatter_start(j, b)
                    @pl.when(j >= _STAG_W)
                    def _():
                        scatter_wait(j - _STAG_W, bn)
                    @pl.when(j <= _NCHUNK - 1 - _STAG_G)
                    def _():
                        gather_start(j + _STAG_G, bn, off)

            scatter_wait(_NCHUNK - 2, (_NCHUNK - 2) % _NBUF)
            scatter_wait(_NCHUNK - 1, (_NCHUNK - 1) % _NBUF)

            plsc.subcore_barrier()

            @pl.when(s < _NS - 1)
            def _():
                pltpu.sync_copy(
                    acc.at[pl.ds(s * 624, 624)],
                    out_hbm.at[qg, pl.ds(s * 624, 624)],
                )

            @pl.when(s == _NS - 1)
            def _():
                pltpu.sync_copy(
                    acc.at[pl.ds(9360, 640)], out_hbm.at[qg, pl.ds(9360, 640)]
                )

    return body


def _make_edge_call(emit_alpha):
    out_type = [jax.ShapeDtypeStruct((_NQ, _N, _Q), jnp.float32)]
    if emit_alpha:
        out_type.append(jax.ShapeDtypeStruct((_E,), jnp.float32))
    scratch = [
        pltpu.VMEM((_N,), jnp.float32),
        pltpu.VMEM((_N,), jnp.float32),
        pltpu.VMEM((_N,), jnp.float32),
        pltpu.VMEM((_NCHUNK, _CH), jnp.int32),
        pltpu.VMEM((_NBUF, _CH), jnp.int32),
        pltpu.VMEM((_NCHUNK, _CH), jnp.int32),
        pltpu.VMEM((_EPT,), jnp.float32),
        pltpu.VMEM((_CH,), jnp.float32),
    ] + [pltpu.VMEM((_CH, _Q), jnp.float32)] * _NBUF + [
        pltpu.VMEM_SHARED((_AR, _Q), jnp.float32),
    ] + [pltpu.SemaphoreType.DMA] * (2 * _NBUF)
    body = _make_edge_body(emit_alpha)

    def call(xs, rows3, cols3, al, ar, dis):
        return pl.kernel(
            body,
            out_type=tuple(out_type) if emit_alpha else out_type[0],
            mesh=_mesh,
            scratch_types=scratch,
            compiler_params=_sc_params,
        )(xs, rows3, cols3, al, ar, dis)

    return call


_edge_plain = _make_edge_call(False)
_edge_alpha = _make_edge_call(True)


def _pre_body(x_ref, wl_ref, wr_ref, degp_ref, al_ref, ar_ref, dis_ref):
    x = x_ref[...]
    al_ref[...] = x @ wl_ref[...]
    ar_ref[...] = x @ wr_ref[...]
    deg = jnp.sum(degp_ref[...], axis=0) + 1.0
    dis_ref[...] = lax.rsqrt(deg)


def _pre(x, wl, wr, degp):
    return pl.pallas_call(
        _pre_body,
        out_shape=(jax.ShapeDtypeStruct((_N,), jnp.float32),) * 3,
    )(x, wl, wr, degp)


_BL = 2000
_NB = _N // _BL


def _layer_body(s_ref, xin_ref, x0_ref, al_ref, ar_ref, dis_ref, wl_ref, wr_ref,
                h_ref, h4_ref, t_ref, aln_ref, arn_ref):
    sc = jnp.concatenate([s_ref[i] for i in range(_NQ)], axis=1)
    t = jnp.tanh(al_ref[0, 0] + ar_ref[0, 0])
    dis = dis_ref[0, 0]
    cf = t * dis * dis
    h = sc + cf[:, None] * xin_ref[...] + _EPS * x0_ref[...]
    h = jnp.maximum(h, 0.0)
    h_ref[...] = h
    for i in range(_NQ):
        h4_ref[i] = h[:, i * _Q:(i + 1) * _Q]
    t_ref[0, 0] = t
    aln_ref[0, 0] = h @ wl_ref[0]
    arn_ref[0, 0] = h @ wr_ref[0]


def _layer(s, xin, x0, al, ar, dis, wl, wr):
    vec_spec = pl.BlockSpec((1, 1, _BL), lambda i: (i, 0, 0))
    mat_spec = pl.BlockSpec((_BL, _D), lambda i: (i, 0))
    h, h4, t, aln, arn = pl.pallas_call(
        _layer_body,
        grid=(_NB,),
        in_specs=[
            pl.BlockSpec((_NQ, _BL, _Q), lambda i: (0, i, 0)),
            mat_spec,
            mat_spec,
            vec_spec,
            vec_spec,
            vec_spec,
            pl.BlockSpec((1, _D), lambda i: (0, 0)),
            pl.BlockSpec((1, _D), lambda i: (0, 0)),
        ],
        out_specs=(
            mat_spec,
            pl.BlockSpec((_NQ, _BL, _Q), lambda i: (0, i, 0)),
            vec_spec,
            vec_spec,
            vec_spec,
        ),
        out_shape=(
            jax.ShapeDtypeStruct((_N, _D), jnp.float32),
            jax.ShapeDtypeStruct((_NQ, _N, _Q), jnp.float32),
            jax.ShapeDtypeStruct((_NB, 1, _BL), jnp.float32),
            jax.ShapeDtypeStruct((_NB, 1, _BL), jnp.float32),
            jax.ShapeDtypeStruct((_NB, 1, _BL), jnp.float32),
        ),
    )(
        s, xin, x0,
        al.reshape(_NB, 1, _BL), ar.reshape(_NB, 1, _BL),
        dis.reshape(_NB, 1, _BL),
        wl.reshape(1, _D), wr.reshape(1, _D),
    )
    return h, h4, t.reshape(_N), aln.reshape(_N), arn.reshape(_N)


def kernel(x, edges_index, pertub, w_l0, w_r0, w_l1, w_r1):
    row = edges_index[0]
    col = edges_index[1]
    rows3 = row.reshape(_NS, _NCHUNK, _CH)
    cols3 = col.reshape(_NS, _NCHUNK, _CH)
    x4 = jnp.concatenate(
        [x[:, i * _Q:(i + 1) * _Q] for i in range(_NQ)], axis=0
    )

    degp = _deg_partials(col)
    al0, ar0, dis = _pre(x, w_l0, w_r0, degp)
    s0 = _edge_plain(x4, rows3, cols3, al0, ar0, dis)
    h1, h14, _t0, al1, ar1 = _layer(s0, x, x, al0, ar0, dis, w_l1, w_r1)
    h14 = h14.reshape(_NQ * _N, _Q)
    s1, alpha_e = _edge_alpha(h14, rows3, cols3, al1, ar1, dis)
    h2, _h24, alpha_n, _a, _b = _layer(s1, h1, x, al1, ar1, dis, w_l1, w_r1)

    loop = jnp.arange(_N, dtype=edges_index.dtype)
    edge_index_sl = jnp.stack(
        [jnp.concatenate([row, loop]), jnp.concatenate([col, loop])]
    )
    alpha = jnp.concatenate([alpha_e, alpha_n])
    return h2, edge_index_sl, alpha
